# baseline (speedup 1.0000x reference)
import functools

import jax
import jax.numpy as jnp
from jax import lax
from jax.experimental import pallas as pl
from jax.experimental.pallas import tpu as pltpu
from jax.experimental.pallas import tpu_sc as plsc

R = 128
N = 32768
NC = 2
NS = 16
NW = NC * NS
RPW = R // NW
L = 16
NV = N // L
UNROLL = 8
STEPS = NV // UNROLL

_mesh = plsc.VectorSubcoreMesh(core_axis_name="c", subcore_axis_name="s")


def _xlane_reduce(x, op):
    for k in (1, 2, 4, 8):
        perm = lax.iota(jnp.int32, L) ^ k
        x = op(x, x.at[perm].get(mode="promise_in_bounds"))
    return x


@functools.partial(
    pl.kernel,
    out_type=jax.ShapeDtypeStruct((NW, L), jnp.int32),
    mesh=_mesh,
    scratch_types=[
        pltpu.VMEM((2, N), jnp.float32),
        pltpu.VMEM((L,), jnp.int32),
        pltpu.SemaphoreType.DMA,
        pltpu.SemaphoreType.DMA,
    ],
)
def _argmax_rows(x_hbm, out_hbm, buf, res, sem0, sem1):
    wid = lax.axis_index("s") * NC + lax.axis_index("c")
    row0 = wid * RPW
    sems = (sem0, sem1)
    lane = lax.iota(jnp.int32, L)

    def start(t):
        slot = t % 2
        return pltpu.async_copy(x_hbm.at[row0 + t], buf.at[slot], sems[slot])

    def scan_row(slot_buf):
        def step(i, carry):
            mval, mstep = carry
            base = i * (UNROLL * L)
            for u in range(UNROLL):
                v = slot_buf[pl.ds(base + u * L, L)]
                p = v > mval
                mval = jnp.where(p, v, mval)
                si = i * UNROLL + u
                mstep = jnp.where(p, jnp.broadcast_to(si, (L,)), mstep)
            return mval, mstep

        init = (jnp.full((L,), -jnp.inf, jnp.float32),
                jnp.zeros((L,), jnp.int32))
        mval, mstep = lax.fori_loop(0, STEPS, step, init)
        midx = mstep * L + lane
        m = _xlane_reduce(mval, jnp.maximum)
        cand = jnp.where(mval == m, midx, jnp.broadcast_to(jnp.int32(N), (L,)))
        return _xlane_reduce(cand, jnp.minimum)

    descs = [start(0)]
    resv = jnp.zeros((L,), jnp.int32)
    for t in range(RPW):
        if t + 1 < RPW:
            descs.append(start(t + 1))
        descs[t].wait()
        r = scan_row(buf.at[t % 2])
        resv = jnp.where(lane == t, jnp.broadcast_to(r, (L,)), resv)
    res[...] = resv
    pltpu.sync_copy(res, out_hbm.at[wid])


def kernel(inputs):
    out = _argmax_rows(inputs)
    return out[:, :RPW].reshape(R).astype(jnp.int64)


# speedup vs baseline: 1.1448x; 1.1448x over previous
import functools

import jax
import jax.numpy as jnp
from jax import lax
from jax.experimental import pallas as pl
from jax.experimental.pallas import tpu as pltpu
from jax.experimental.pallas import tpu_sc as plsc

R = 128
N = 32768
NC = 2
NS = 16
NW = NC * NS
RPW = R // NW
L = 16
NV = N // L
UNROLL = 8
STEPS = NV // UNROLL

_mesh = plsc.VectorSubcoreMesh(core_axis_name="c", subcore_axis_name="s",
                               num_cores=NC, num_subcores=NS)


def _xlane_reduce(x, op):
    for k in (1, 2, 4, 8):
        perm = lax.iota(jnp.int32, L) ^ k
        x = op(x, x.at[perm].get(mode="promise_in_bounds"))
    return x


@functools.partial(
    pl.kernel,
    out_type=jax.ShapeDtypeStruct((NW, L), jnp.int32),
    mesh=_mesh,
    scratch_types=[
        pltpu.VMEM((2, N), jnp.float32),
        pltpu.VMEM((L,), jnp.int32),
        pltpu.SemaphoreType.DMA,
        pltpu.SemaphoreType.DMA,
    ],
)
def _argmax_rows(x_hbm, out_hbm, buf, res, sem0, sem1):
    wid = lax.axis_index("s") * NC + lax.axis_index("c")
    row0 = wid * RPW
    sems = (sem0, sem1)
    lane = lax.iota(jnp.int32, L)

    def start(t):
        slot = t % 2
        return pltpu.async_copy(x_hbm.at[row0 + t], buf.at[slot], sems[slot])

    def scan_row(slot_buf):
        def step(i, carry):
            mvals, msteps = carry
            base = i * (UNROLL * L)
            ibc = jnp.broadcast_to(i, (L,))
            mvals, msteps = list(mvals), list(msteps)
            for u in range(UNROLL):
                v = slot_buf[pl.ds(base + u * L, L)]
                p = v > mvals[u]
                mvals[u] = jnp.where(p, v, mvals[u])
                msteps[u] = jnp.where(p, ibc, msteps[u])
            return tuple(mvals), tuple(msteps)

        init = (tuple(jnp.full((L,), -jnp.inf, jnp.float32)
                      for _ in range(UNROLL)),
                tuple(jnp.zeros((L,), jnp.int32) for _ in range(UNROLL)))
        mvals, msteps = lax.fori_loop(0, STEPS, step, init)

        pairs = [(mvals[u], (msteps[u] * UNROLL + u) * L + lane)
                 for u in range(UNROLL)]
        while len(pairs) > 1:
            nxt = []
            for j in range(0, len(pairs), 2):
                (av, ai), (bv, bi) = pairs[j], pairs[j + 1]
                p = (av > bv) | ((av == bv) & (ai < bi))
                nxt.append((jnp.where(p, av, bv), jnp.where(p, ai, bi)))
            pairs = nxt
        mval, midx = pairs[0]
        m = _xlane_reduce(mval, jnp.maximum)
        cand = jnp.where(mval == m, midx, jnp.broadcast_to(jnp.int32(N), (L,)))
        return _xlane_reduce(cand, jnp.minimum)

    descs = [start(0)]
    resv = jnp.zeros((L,), jnp.int32)
    for t in range(RPW):
        if t + 1 < RPW:
            descs.append(start(t + 1))
        descs[t].wait()
        r = scan_row(buf.at[t % 2])
        resv = jnp.where(lane == t, jnp.broadcast_to(r, (L,)), resv)
    res[...] = resv
    pltpu.sync_copy(res, out_hbm.at[wid])


def kernel(inputs):
    out = _argmax_rows(inputs)
    return out[:, :RPW].reshape(R).astype(jnp.int64)
